# Initial kernel scaffold; baseline (speedup 1.0000x reference)
import jax
import jax.numpy as jnp
from jax.experimental import pallas as pl


def kernel(half_edge_features, neighborhoods, conv_w, conv_b):
    raise NotImplementedError("write your pallas kernel here")


# speedup vs baseline: 5.2674x; 5.2674x over previous
import functools

import jax
import jax.numpy as jnp
from jax import lax
from jax.experimental import pallas as pl
from jax.experimental.pallas import tpu as pltpu
from jax.experimental.pallas import tpu_sc as plsc

C_IN = 128
C_OUT = 128
HE = 160000
KW = 5

NC, NS = 2, 16
NW = NC * NS
CHUNK = 128
HE_PAD = 163840
EPW = HE_PAD // NW
NCH = EPW // CHUNK

E_BLK = 3200


def _p1_body(f_ref, w_ref, b_ref, y0_ref, z1_ref, z2_ref, z3_ref, z4_ref):
    f = f_ref[...]
    y0_ref[...] = lax.dot_general(
        w_ref[0], f, (((1,), (0,)), ((), ())),
        preferred_element_type=jnp.float32) + b_ref[...]
    for k, z_ref in ((1, z1_ref), (2, z2_ref), (3, z3_ref), (4, z4_ref)):
        z_ref[...] = lax.dot_general(
            f, w_ref[k], (((0,), (1,)), ((), ())),
            preferred_element_type=jnp.float32)


def _pass1(f2d, w, b2):
    zt_shape = jax.ShapeDtypeStruct((HE, C_OUT), jnp.float32)
    return pl.pallas_call(
        _p1_body,
        grid=(HE // E_BLK,),
        in_specs=[
            pl.BlockSpec((C_IN, E_BLK), lambda i: (0, i)),
            pl.BlockSpec((KW, C_OUT, C_IN), lambda i: (0, 0, 0)),
            pl.BlockSpec((C_OUT, 1), lambda i: (0, 0)),
        ],
        out_specs=[
            pl.BlockSpec((C_OUT, E_BLK), lambda i: (0, i)),
            pl.BlockSpec((E_BLK, C_OUT), lambda i: (i, 0)),
            pl.BlockSpec((E_BLK, C_OUT), lambda i: (i, 0)),
            pl.BlockSpec((E_BLK, C_OUT), lambda i: (i, 0)),
            pl.BlockSpec((E_BLK, C_OUT), lambda i: (i, 0)),
        ],
        out_shape=[jax.ShapeDtypeStruct((C_OUT, HE), jnp.float32),
                   zt_shape, zt_shape, zt_shape, zt_shape],
    )(f2d, w, b2)


def _sc_body(z1, z2, z3, z4, nbh_flat, s_out,
             i0, i1, i2, i3, r0, r1, r2, r3, sem):
    wid = lax.axis_index("s") * NC + lax.axis_index("c")
    base = wid * EPW

    def chunk_body(c, _):
        off = base + c * CHUNK
        pltpu.sync_copy(nbh_flat.at[pl.ds(0 * HE_PAD + off, CHUNK)], i0)
        pltpu.sync_copy(nbh_flat.at[pl.ds(1 * HE_PAD + off, CHUNK)], i1)
        pltpu.sync_copy(nbh_flat.at[pl.ds(2 * HE_PAD + off, CHUNK)], i2)
        pltpu.sync_copy(nbh_flat.at[pl.ds(3 * HE_PAD + off, CHUNK)], i3)
        c0 = pltpu.async_copy(z1.at[i0], r0, sem)
        c1 = pltpu.async_copy(z2.at[i1], r1, sem)
        c2 = pltpu.async_copy(z3.at[i2], r2, sem)
        c3 = pltpu.async_copy(z4.at[i3], r3, sem)
        c0.wait()
        c1.wait()
        c2.wait()
        c3.wait()

        def row_body(rr, _):
            for j in range(C_OUT // 16):
                s = pl.ds(j * 16, 16)
                r0[rr, s] = (r0[rr, s] + r1[rr, s]) + (r2[rr, s] + r3[rr, s])
            return 0

        lax.fori_loop(0, CHUNK, row_body, 0)
        pltpu.sync_copy(r0, s_out.at[pl.ds(off, CHUNK)])
        return 0

    lax.fori_loop(0, NCH, chunk_body, 0)


_sc_gather_sum = functools.partial(
    pl.kernel,
    out_type=jax.ShapeDtypeStruct((HE_PAD, C_OUT), jnp.float32),
    mesh=plsc.VectorSubcoreMesh(core_axis_name="c", subcore_axis_name="s"),
    scratch_types=[
        pltpu.VMEM((CHUNK,), jnp.int32),
        pltpu.VMEM((CHUNK,), jnp.int32),
        pltpu.VMEM((CHUNK,), jnp.int32),
        pltpu.VMEM((CHUNK,), jnp.int32),
        pltpu.VMEM((CHUNK, C_OUT), jnp.float32),
        pltpu.VMEM((CHUNK, C_OUT), jnp.float32),
        pltpu.VMEM((CHUNK, C_OUT), jnp.float32),
        pltpu.VMEM((CHUNK, C_OUT), jnp.float32),
        pltpu.SemaphoreType.DMA,
    ],
)(_sc_body)


def _p3_body(y0_ref, s_ref, o_ref):
    o_ref[...] = y0_ref[...] + s_ref[...].T


def _pass3(y0, s):
    return pl.pallas_call(
        _p3_body,
        grid=(HE // E_BLK,),
        in_specs=[
            pl.BlockSpec((C_OUT, E_BLK), lambda i: (0, i)),
            pl.BlockSpec((E_BLK, C_OUT), lambda i: (i, 0)),
        ],
        out_specs=pl.BlockSpec((C_OUT, E_BLK), lambda i: (0, i)),
        out_shape=jax.ShapeDtypeStruct((C_OUT, HE), jnp.float32),
    )(y0, s)


def kernel(half_edge_features, neighborhoods, conv_w, conv_b):
    f2d = half_edge_features[0]
    w = jnp.transpose(conv_w[:, :, 0, :], (2, 0, 1))
    b2 = conv_b[:, None]

    y0, z1, z2, z3, z4 = _pass1(f2d, w, b2)

    nbh_t = jnp.transpose(neighborhoods[0])
    nbh_flat = jnp.pad(nbh_t, ((0, 0), (0, HE_PAD - HE))).reshape(-1)

    s = _sc_gather_sum(z1, z2, z3, z4, nbh_flat)

    out = _pass3(y0, s)
    return out[None, :, :, None]


# speedup vs baseline: 8.2669x; 1.5694x over previous
import functools

import jax
import jax.numpy as jnp
from jax import lax
from jax.experimental import pallas as pl
from jax.experimental.pallas import tpu as pltpu
from jax.experimental.pallas import tpu_sc as plsc

C_IN = 128
C_OUT = 128
HE = 160000
KW = 5

NC, NS = 2, 16
NW = NC * NS
CHUNK = 64
HE_PAD = 163840
EPW = HE_PAD // NW
NCH = EPW // CHUNK
NPAIR = NCH // 2

E_BLK = 3200


def _p1_body(f_ref, w_ref, b_ref, y0_ref, z1_ref, z2_ref, z3_ref, z4_ref):
    f = f_ref[...]
    y0_ref[...] = lax.dot_general(
        w_ref[0], f, (((1,), (0,)), ((), ())),
        preferred_element_type=jnp.float32) + b_ref[...]
    for k, z_ref in ((1, z1_ref), (2, z2_ref), (3, z3_ref), (4, z4_ref)):
        z_ref[...] = lax.dot_general(
            f, w_ref[k], (((0,), (1,)), ((), ())),
            preferred_element_type=jnp.float32)


def _pass1(f2d, w, b2):
    zt_shape = jax.ShapeDtypeStruct((HE, C_OUT), jnp.float32)
    return pl.pallas_call(
        _p1_body,
        grid=(HE // E_BLK,),
        in_specs=[
            pl.BlockSpec((C_IN, E_BLK), lambda i: (0, i)),
            pl.BlockSpec((KW, C_OUT, C_IN), lambda i: (0, 0, 0)),
            pl.BlockSpec((C_OUT, 1), lambda i: (0, 0)),
        ],
        out_specs=[
            pl.BlockSpec((C_OUT, E_BLK), lambda i: (0, i)),
            pl.BlockSpec((E_BLK, C_OUT), lambda i: (i, 0)),
            pl.BlockSpec((E_BLK, C_OUT), lambda i: (i, 0)),
            pl.BlockSpec((E_BLK, C_OUT), lambda i: (i, 0)),
            pl.BlockSpec((E_BLK, C_OUT), lambda i: (i, 0)),
        ],
        out_shape=[jax.ShapeDtypeStruct((C_OUT, HE), jnp.float32),
                   zt_shape, zt_shape, zt_shape, zt_shape],
    )(f2d, w, b2)


def _sc_body(z1, z2, z3, z4, nbh4, s_out,
             i0, i1, i2, i3,
             g0a, g1a, g2a, g3a, g0b, g1b, g2b, g3b,
             acc_a, acc_b, gsa, gsb, ssa, ssb):
    wid = lax.axis_index("s") * NC + lax.axis_index("c")
    base = wid * EPW
    zs = (z1, z2, z3, z4)
    idxs = (i0, i1, i2, i3)
    slots = (
        ((g0a, g1a, g2a, g3a), acc_a, gsa, ssa),
        ((g0b, g1b, g2b, g3b), acc_b, gsb, ssb),
    )

    for k in range(4):
        pltpu.sync_copy(nbh4.at[k, wid], idxs[k])

    def fire_g(c, slot):
        gbufs, _, gsem, _ = slots[slot]
        for k in range(4):
            pltpu.make_async_copy(zs[k].at[idxs[k].at[c]], gbufs[k], gsem).start()

    def wait_g(slot):
        gbufs, _, gsem, _ = slots[slot]
        for k in range(4):
            pltpu.make_async_copy(zs[k].at[idxs[k].at[0]], gbufs[k], gsem).wait()

    def fire_store(c, slot):
        _, acc, _, ssem = slots[slot]
        off = base + c * CHUNK
        pltpu.make_async_copy(acc, s_out.at[pl.ds(off, CHUNK)], ssem).start()

    def wait_store(slot):
        _, acc, _, ssem = slots[slot]
        pltpu.make_async_copy(acc, s_out.at[pl.ds(base, CHUNK)], ssem).wait()

    def accum(slot):
        gbufs, acc, _, _ = slots[slot]
        g0, g1, g2, g3 = gbufs

        def row_body(rr, _):
            for j in range(C_OUT // 16):
                s = pl.ds(j * 16, 16)
                acc[rr, s] = (g0[rr, s] + g1[rr, s]) + (g2[rr, s] + g3[rr, s])
            return 0

        lax.fori_loop(0, CHUNK, row_body, 0)

    fire_g(0, 0)

    def pair_body(p, _):
        c0 = 2 * p
        fire_g(c0 + 1, 1)
        wait_g(0)

        @pl.when(p >= 1)
        def _():
            wait_store(0)

        accum(0)
        fire_store(c0, 0)

        @pl.when(p < NPAIR - 1)
        def _():
            fire_g(c0 + 2, 0)

        wait_g(1)

        @pl.when(p >= 1)
        def _():
            wait_store(1)

        accum(1)
        fire_store(c0 + 1, 1)
        return 0

    lax.fori_loop(0, NPAIR, pair_body, 0)
    wait_store(0)
    wait_store(1)


_GBUF = pltpu.VMEM((CHUNK, C_OUT), jnp.float32)
_sc_gather_sum = functools.partial(
    pl.kernel,
    out_type=jax.ShapeDtypeStruct((HE_PAD, C_OUT), jnp.float32),
    mesh=plsc.VectorSubcoreMesh(core_axis_name="c", subcore_axis_name="s"),
    scratch_types=[
        pltpu.VMEM((NCH, CHUNK), jnp.int32),
        pltpu.VMEM((NCH, CHUNK), jnp.int32),
        pltpu.VMEM((NCH, CHUNK), jnp.int32),
        pltpu.VMEM((NCH, CHUNK), jnp.int32),
        _GBUF, _GBUF, _GBUF, _GBUF, _GBUF, _GBUF, _GBUF, _GBUF,
        _GBUF, _GBUF,
        pltpu.SemaphoreType.DMA,
        pltpu.SemaphoreType.DMA,
        pltpu.SemaphoreType.DMA,
        pltpu.SemaphoreType.DMA,
    ],
)(_sc_body)


def _p3_body(y0_ref, s_ref, o_ref):
    o_ref[...] = y0_ref[...] + s_ref[...].T


def _pass3(y0, s):
    return pl.pallas_call(
        _p3_body,
        grid=(HE // E_BLK,),
        in_specs=[
            pl.BlockSpec((C_OUT, E_BLK), lambda i: (0, i)),
            pl.BlockSpec((E_BLK, C_OUT), lambda i: (i, 0)),
        ],
        out_specs=pl.BlockSpec((C_OUT, E_BLK), lambda i: (0, i)),
        out_shape=jax.ShapeDtypeStruct((C_OUT, HE), jnp.float32),
    )(y0, s)


def kernel(half_edge_features, neighborhoods, conv_w, conv_b):
    f2d = half_edge_features[0]
    w = jnp.transpose(conv_w[:, :, 0, :], (2, 0, 1))
    b2 = conv_b[:, None]

    y0, z1, z2, z3, z4 = _pass1(f2d, w, b2)

    nbh_t = jnp.transpose(neighborhoods[0])
    nbh4 = jnp.pad(nbh_t, ((0, 0), (0, HE_PAD - HE))).reshape(
        KW - 1, NW, NCH, CHUNK)

    s = _sc_gather_sum(z1, z2, z3, z4, nbh4)

    out = _pass3(y0, s)
    return out[None, :, :, None]
